# baseline (speedup 1.0000x reference)
import jax
import jax.numpy as jnp
from jax.experimental import pallas as pl


def _gat_dense_kernel(data_ref, w_ref, asrc_ref, adst_ref, bias_ref, out_ref):
    n = data_ref.shape[0]
    xp = jnp.dot(data_ref[:], w_ref[:], preferred_element_type=jnp.float32)
    a_s = jax.lax.dot_general(
        asrc_ref[:], xp, (((1,), (1,)), ((), ())),
        preferred_element_type=jnp.float32)
    a_d = jnp.dot(xp, adst_ref[:], preferred_element_type=jnp.float32)
    e = a_d + a_s
    e = jnp.where(e > 0, e, 0.2 * e)
    row = jax.lax.broadcasted_iota(jnp.int32, (n, n), 0)
    col = jax.lax.broadcasted_iota(jnp.int32, (n, n), 1)
    mask = col <= row
    e = jnp.where(mask, e, -1e30)
    m = jnp.max(e, axis=1, keepdims=True)
    ex = jnp.where(mask, jnp.exp(e - m), 0.0)
    denom = jnp.sum(ex, axis=1, keepdims=True)
    alpha = ex / denom
    out = jnp.dot(alpha, xp, preferred_element_type=jnp.float32) + bias_ref[:]
    out_ref[:] = jnp.maximum(out, 0.0)


def kernel(data, W, att_src, att_dst, bias):
    n, out_ch = data.shape[0], W.shape[1]
    return pl.pallas_call(
        _gat_dense_kernel,
        out_shape=jax.ShapeDtypeStruct((n, out_ch), jnp.float32),
    )(
        data,
        W,
        att_src.reshape(1, out_ch),
        att_dst.reshape(out_ch, 1),
        bias.reshape(1, out_ch),
    )
